# Initial kernel scaffold; baseline (speedup 1.0000x reference)
import jax
import jax.numpy as jnp
from jax.experimental import pallas as pl


def kernel(x, edge_index, W1, a_src1, a_dst1, b1, gamma, beta, run_mean, run_var, W2, a_src2, a_dst2, b2):
    raise NotImplementedError("write your pallas kernel here")


# speedup vs baseline: 1.0001x; 1.0001x over previous
import jax
import jax.numpy as jnp
from jax.experimental import pallas as pl

N = 10000
HEADS = 8
HID = 16
D_OUT = 64
NEG_SLOPE = 0.2


def _segment_softmax(alpha, dst, num_nodes):
    m = jax.ops.segment_max(alpha, dst, num_segments=num_nodes)
    alpha = jnp.exp(alpha - m[dst])
    s = jax.ops.segment_sum(alpha, dst, num_segments=num_nodes)
    return alpha / (s[dst] + 1e-16)


def _gat_conv(x, edge_index, W, a_src, a_dst, b, heads, ch, concat, neg_slope):
    num_nodes = x.shape[0]
    ar = jnp.arange(num_nodes, dtype=edge_index.dtype)
    ei = jnp.concatenate([edge_index, jnp.stack([ar, ar])], axis=1)
    src, dst = ei[0], ei[1]
    h = (x @ W).reshape(num_nodes, heads, ch)
    alpha_s = jnp.sum(h * a_src[None, :, :], axis=-1)
    alpha_d = jnp.sum(h * a_dst[None, :, :], axis=-1)
    alpha = alpha_s[src] + alpha_d[dst]
    alpha = jnp.where(alpha >= 0, alpha, neg_slope * alpha)
    alpha = _segment_softmax(alpha, dst, num_nodes)
    msg = h[src] * alpha[:, :, None]
    out = jax.ops.segment_sum(msg, dst, num_segments=num_nodes)
    if concat:
        out = out.reshape(num_nodes, heads * ch)
    else:
        out = out.mean(axis=1)
    return out + b


def _log_softmax_kernel(x_ref, o_ref):
    x = x_ref[...]
    m = jnp.max(x, axis=-1, keepdims=True)
    e = jnp.exp(x - m)
    s = jnp.sum(e, axis=-1, keepdims=True)
    o_ref[...] = (x - m) - jnp.log(s)


def kernel(x, edge_index, W1, a_src1, a_dst1, b1, gamma, beta, run_mean,
           run_var, W2, a_src2, a_dst2, b2):
    h = _gat_conv(x, edge_index, W1, a_src1, a_dst1, b1, HEADS, HID, True, NEG_SLOPE)
    h = (h - run_mean) / jnp.sqrt(run_var + 1e-5) * gamma + beta
    h = jax.nn.relu(h)
    h = _gat_conv(h, edge_index, W2, a_src2, a_dst2, b2, 1, D_OUT, False, NEG_SLOPE)
    out = pl.pallas_call(
        _log_softmax_kernel,
        out_shape=jax.ShapeDtypeStruct((N, D_OUT), jnp.float32),
        grid=(10,),
        in_specs=[pl.BlockSpec((N // 10, D_OUT), lambda i: (i, 0))],
        out_specs=pl.BlockSpec((N // 10, D_OUT), lambda i: (i, 0)),
    )(h)
    return out


# speedup vs baseline: 44.1244x; 44.1187x over previous
import functools

import jax
import jax.numpy as jnp
from jax import lax
from jax.experimental import pallas as pl
from jax.experimental.pallas import tpu as pltpu
from jax.experimental.pallas import tpu_sc as plsc

N = 10000
NR = 10240
D_IN = 128
HEADS = 8
HID = 16
D_OUT = 64
NEG = 0.2
ROW1 = 144
ROW2 = 80
CHK = 128
NW = 32
BLK = 512
NEG_BIG = -1e30


def _stage1_body(x_ref, w_ref, a_ref, o_ref):
    i = pl.program_id(0)
    h = jnp.dot(x_ref[...], w_ref[...], preferred_element_type=jnp.float32)
    asd = jnp.dot(h, a_ref[...], preferred_element_type=jnp.float32)
    rows = i * BLK + lax.broadcasted_iota(jnp.int32, (BLK, 1), 0)
    valid = rows < N
    hm = jnp.where(valid, h, 0.0)
    asm = jnp.where(valid, asd[:, :8], NEG_BIG)
    adm = jnp.where(valid, asd[:, 8:], 0.0)
    o_ref[:, 0:128] = hm
    o_ref[:, 128:144] = jnp.concatenate([asm, adm], axis=1)


def _stage1(xp, W1, A1):
    return pl.pallas_call(
        _stage1_body,
        out_shape=jax.ShapeDtypeStruct((NR, ROW1), jnp.float32),
        grid=(NR // BLK,),
        in_specs=[
            pl.BlockSpec((BLK, D_IN), lambda i: (i, 0)),
            pl.BlockSpec((D_IN, D_IN), lambda i: (0, 0)),
            pl.BlockSpec((D_IN, 16), lambda i: (0, 0)),
        ],
        out_specs=pl.BlockSpec((BLK, ROW1), lambda i: (i, 0)),
    )(xp, W1, A1)


def _shift1_body(t_ref, o_ref):
    t = t_ref[:, 128:144]
    asv, adv = t[:, :8], t[:, 8:]
    m = jnp.max(asv, axis=0, keepdims=True)
    tt = m + adv
    c = jnp.where(tt >= 0, tt, NEG * tt)
    z = jnp.zeros_like(adv)
    o_ref[...] = jnp.concatenate([adv, z, c, z], axis=1)


def _shift1(table1):
    return pl.pallas_call(
        _shift1_body,
        out_shape=jax.ShapeDtypeStruct((NR, 32), jnp.float32),
        grid=(1,),
        in_specs=[pl.BlockSpec((NR, ROW1), lambda i: (0, 0))],
        out_specs=pl.BlockSpec((NR, 32), lambda i: (0, 0)),
    )(table1)


def _stage2_body(p0_ref, p1_ref, w_ref, a_ref, r_ref, b1_ref, sc_ref, sh_ref,
                 o_ref):
    i = pl.program_id(0)
    s = p0_ref[...] + p1_ref[...]
    feats = s[:, :128]
    den = s[:, 128:136]
    denb = jnp.dot(den, r_ref[...], preferred_element_type=jnp.float32)
    h1 = feats / (denb + 1e-16) + b1_ref[...]
    h1 = h1 * sc_ref[...] + sh_ref[...]
    h1 = jnp.maximum(h1, 0.0)
    h2 = jnp.dot(h1, w_ref[...], preferred_element_type=jnp.float32)
    asd = jnp.dot(h2, a_ref[...], preferred_element_type=jnp.float32)
    rows = i * BLK + lax.broadcasted_iota(jnp.int32, (BLK, 1), 0)
    valid = rows < N
    padrow = jnp.where(
        lax.broadcasted_iota(jnp.int32, (1, 16), 1) == 0, NEG_BIG, 0.0)
    o_ref[:, 0:64] = jnp.where(valid, h2, 0.0)
    o_ref[:, 64:80] = jnp.where(valid, asd, padrow)


def _stage2(p0, p1, W2, A2, R, b1, scale, shift):
    return pl.pallas_call(
        _stage2_body,
        out_shape=jax.ShapeDtypeStruct((NR, ROW2), jnp.float32),
        grid=(NR // BLK,),
        in_specs=[
            pl.BlockSpec((BLK, ROW1), lambda i: (i, 0)),
            pl.BlockSpec((BLK, ROW1), lambda i: (i, 0)),
            pl.BlockSpec((128, 64), lambda i: (0, 0)),
            pl.BlockSpec((64, 16), lambda i: (0, 0)),
            pl.BlockSpec((8, 128), lambda i: (0, 0)),
            pl.BlockSpec((1, 128), lambda i: (0, 0)),
            pl.BlockSpec((1, 128), lambda i: (0, 0)),
            pl.BlockSpec((1, 128), lambda i: (0, 0)),
        ],
        out_specs=pl.BlockSpec((BLK, ROW2), lambda i: (i, 0)),
    )(p0, p1, W2, A2, R, b1, scale, shift)


def _shift2_body(t_ref, o_ref):
    t = t_ref[:, 64:80]
    asv, adv = t[:, 0:1], t[:, 1:2]
    m = jnp.max(asv, axis=0, keepdims=True)
    tt = m + adv
    c = jnp.where(tt >= 0, tt, NEG * tt)
    z = jnp.zeros((t.shape[0], 15), dtype=jnp.float32)
    o_ref[...] = jnp.concatenate([adv, z, c, z], axis=1)


def _shift2(table2):
    return pl.pallas_call(
        _shift2_body,
        out_shape=jax.ShapeDtypeStruct((NR, 32), jnp.float32),
        grid=(1,),
        in_specs=[pl.BlockSpec((NR, ROW2), lambda i: (0, 0))],
        out_specs=pl.BlockSpec((NR, 32), lambda i: (0, 0)),
    )(table2)


def _stage3_body(q0_ref, q1_ref, b2_ref, o_ref):
    s = q0_ref[...] + q1_ref[...]
    feats = s[:, :64]
    den = s[:, 64:65]
    o = feats / (den + 1e-16) + b2_ref[...]
    m = jnp.max(o, axis=1, keepdims=True)
    l = o - m
    o_ref[...] = l - jnp.log(jnp.sum(jnp.exp(l), axis=1, keepdims=True))


def _stage3(q0, q1, b2):
    return pl.pallas_call(
        _stage3_body,
        out_shape=jax.ShapeDtypeStruct((NR, D_OUT), jnp.float32),
        grid=(NR // BLK,),
        in_specs=[
            pl.BlockSpec((BLK, ROW2), lambda i: (i, 0)),
            pl.BlockSpec((BLK, ROW2), lambda i: (i, 0)),
            pl.BlockSpec((1, 64), lambda i: (0, 0)),
        ],
        out_specs=pl.BlockSpec((BLK, D_OUT), lambda i: (i, 0)),
    )(q0, q1, b2)


def _make_edge_pass(roww, ch, mask_width, ep):
    per_w = ep // NW
    nch = per_w // CHK
    nfeat = (roww - 16) // 16
    zslices = (NR // 16) // CHK
    mesh = plsc.VectorSubcoreMesh(core_axis_name="c", subcore_axis_name="s")

    @functools.partial(
        pl.kernel, mesh=mesh,
        compiler_params=pltpu.CompilerParams(use_tc_tiling_on_sc=False),
        out_type=jax.ShapeDtypeStruct((2, NR, roww), jnp.float32),
        scratch_types=[
            pltpu.VMEM((CHK,), jnp.int32),
            pltpu.VMEM((CHK,), jnp.int32),
            pltpu.VMEM((CHK, roww), jnp.float32),
            pltpu.VMEM((CHK, 32), jnp.float32),
            pltpu.VMEM_SHARED((NR, roww), jnp.float32),
            pltpu.SemaphoreType.DMA,
            pltpu.SemaphoreType.DMA,
        ],
    )
    def edge_pass(t1, tA, srcr, dstr, out, src_v, dst_v, gbuf, abuf,
                  acc, sem1, sem2):
        cid = lax.axis_index("c")
        sid = lax.axis_index("s")
        wid = sid * 2 + cid
        lane = lax.iota(jnp.int32, 16)
        emask = lane < mask_width
        zero16 = jnp.zeros((16,), jnp.float32)

        def zrow(i, _):
            for j in range(roww // 16):
                gbuf[i, pl.ds(j * 16, 16)] = zero16
            return 0
        lax.fori_loop(0, CHK, zrow, 0)
        base = sid * (NR // 16)

        def zcp(k, _):
            pltpu.sync_copy(gbuf, acc.at[pl.ds(base + k * CHK, CHK)])
            return 0
        lax.fori_loop(0, zslices, zcp, 0)
        plsc.subcore_barrier()

        ebase = wid * per_w

        def chunk(k, _):
            off = ebase + k * CHK
            pltpu.sync_copy(srcr.at[pl.ds(off, CHK)], src_v)
            pltpu.sync_copy(dstr.at[pl.ds(off, CHK)], dst_v)
            cp1 = pltpu.async_copy(t1.at[src_v], gbuf, sem1)
            cp2 = pltpu.async_copy(tA.at[dst_v], abuf, sem2)
            cp1.wait()
            cp2.wait()

            def edge(i, _):
                tail = gbuf[i, pl.ds(roww - 16, 16)]
                adv = abuf[i, pl.ds(0, 16)]
                cv = abuf[i, pl.ds(16, 16)]
                raw = tail + adv
                lk = jnp.where(raw >= 0, raw, NEG * raw)
                ev = jnp.exp(lk - cv)
                for j in range(nfeat):
                    hj = (j * 16) // ch
                    ej = lax.gather(
                        ev, jnp.full((16, 1), hj, dtype=jnp.int32),
                        lax.GatherDimensionNumbers(
                            offset_dims=(), collapsed_slice_dims=(0,),
                            start_index_map=(0,)),
                        (1,), mode=lax.GatherScatterMode.PROMISE_IN_BOUNDS)
                    gbuf[i, pl.ds(j * 16, 16)] = gbuf[i, pl.ds(j * 16, 16)] * ej
                gbuf[i, pl.ds(roww - 16, 16)] = jnp.where(emask, ev, zero16)
                return 0
            lax.fori_loop(0, CHK, edge, 0)
            pltpu.sync_copy(gbuf, acc.at[dst_v], add=True)
            return 0
        lax.fori_loop(0, nch, chunk, 0)
        plsc.subcore_barrier()

        def ocp(k, _):
            pltpu.sync_copy(acc.at[pl.ds(base + k * CHK, CHK)],
                            out.at[cid, pl.ds(base + k * CHK, CHK)])
            return 0
        lax.fori_loop(0, zslices, ocp, 0)

    return edge_pass


def kernel(x, edge_index, W1, a_src1, a_dst1, b1, gamma, beta, run_mean,
           run_var, W2, a_src2, a_dst2, b2):
    e2 = edge_index.shape[1] + N
    ep = ((e2 + NW * CHK - 1) // (NW * CHK)) * (NW * CHK)

    xp = jnp.pad(x, ((0, NR - N), (0, 0)))
    ar = jnp.arange(N, dtype=edge_index.dtype)
    ei = jnp.concatenate([edge_index, jnp.stack([ar, ar])], axis=1)
    src = jnp.pad(ei[0], (0, ep - e2), constant_values=N)
    dst = jnp.pad(ei[1], (0, ep - e2), constant_values=N)

    rows128 = jnp.arange(128)
    head_of = jnp.repeat(jnp.arange(8), 16)
    A_src = jnp.zeros((128, 8), jnp.float32).at[rows128, head_of].set(
        a_src1.reshape(128))
    A_dst = jnp.zeros((128, 8), jnp.float32).at[rows128, head_of].set(
        a_dst1.reshape(128))
    A1 = jnp.concatenate([A_src, A_dst], axis=1)
    A2 = jnp.concatenate(
        [a_src2.reshape(64, 1), a_dst2.reshape(64, 1),
         jnp.zeros((64, 14), jnp.float32)], axis=1)
    Rm = jnp.repeat(jnp.eye(8, dtype=jnp.float32), 16, axis=1)
    scale = (gamma / jnp.sqrt(run_var + 1e-5)).reshape(1, 128)
    shift = (beta - run_mean * scale[0]).reshape(1, 128)

    table1 = _stage1(xp, W1, A1)
    tA1 = _shift1(table1)
    acc1 = _make_edge_pass(ROW1, HID, HEADS, ep)(table1, tA1, src, dst)
    table2 = _stage2(acc1[0], acc1[1], W2, A2, Rm, b1.reshape(1, 128),
                     scale, shift)
    tA2 = _shift2(table2)
    acc2 = _make_edge_pass(ROW2, D_OUT, 1, ep)(table2, tA2, src, dst)
    out = _stage3(acc2[0], acc2[1], b2.reshape(1, 64))
    return out[:N]


# speedup vs baseline: 64.2325x; 1.4557x over previous
import functools

import jax
import jax.numpy as jnp
from jax import lax
from jax.experimental import pallas as pl
from jax.experimental.pallas import tpu as pltpu
from jax.experimental.pallas import tpu_sc as plsc

N = 10000
NR = 10240
D_IN = 128
HEADS = 8
HID = 16
D_OUT = 64
NEG = 0.2
ROW1 = 144
ROW2 = 80
CHK = 64
GRP = 54
NW = 32
BLK = 512
NEG_BIG = -1e30


def _stage1_body(x_ref, w_ref, a_ref, o_ref):
    i = pl.program_id(0)
    h = jnp.dot(x_ref[...], w_ref[...], preferred_element_type=jnp.float32)
    asd = jnp.dot(h, a_ref[...], preferred_element_type=jnp.float32)
    rows = i * BLK + lax.broadcasted_iota(jnp.int32, (BLK, 1), 0)
    valid = rows < N
    hm = jnp.where(valid, h, 0.0)
    asm = jnp.where(valid, asd[:, :8], NEG_BIG)
    adm = jnp.where(valid, asd[:, 8:], 0.0)
    o_ref[:, 0:128] = hm
    o_ref[:, 128:144] = jnp.concatenate([asm, adm], axis=1)


def _stage1(xp, W1, A1):
    return pl.pallas_call(
        _stage1_body,
        out_shape=jax.ShapeDtypeStruct((NR, ROW1), jnp.float32),
        grid=(NR // BLK,),
        in_specs=[
            pl.BlockSpec((BLK, D_IN), lambda i: (i, 0)),
            pl.BlockSpec((D_IN, D_IN), lambda i: (0, 0)),
            pl.BlockSpec((D_IN, 16), lambda i: (0, 0)),
        ],
        out_specs=pl.BlockSpec((BLK, ROW1), lambda i: (i, 0)),
    )(xp, W1, A1)


def _shift1_body(t_ref, o_ref):
    t = t_ref[:, 128:144]
    asv, adv = t[:, :8], t[:, 8:]
    m = jnp.max(asv, axis=0, keepdims=True)
    tt = m + adv
    c = jnp.where(tt >= 0, tt, NEG * tt)
    z = jnp.zeros_like(adv)
    o_ref[...] = jnp.concatenate([adv, z, c, z], axis=1)


def _shift1(table1):
    return pl.pallas_call(
        _shift1_body,
        out_shape=jax.ShapeDtypeStruct((NR, 32), jnp.float32),
        grid=(1,),
        in_specs=[pl.BlockSpec((NR, ROW1), lambda i: (0, 0))],
        out_specs=pl.BlockSpec((NR, 32), lambda i: (0, 0)),
    )(table1)


def _stage2_body(p0_ref, p1_ref, w_ref, a_ref, r_ref, b1_ref, sc_ref, sh_ref,
                 o_ref):
    i = pl.program_id(0)
    s = p0_ref[...] + p1_ref[...]
    feats = s[:, :128]
    den = s[:, 128:136]
    denb = jnp.dot(den, r_ref[...], preferred_element_type=jnp.float32)
    h1 = feats / (denb + 1e-16) + b1_ref[...]
    h1 = h1 * sc_ref[...] + sh_ref[...]
    h1 = jnp.maximum(h1, 0.0)
    h2 = jnp.dot(h1, w_ref[...], preferred_element_type=jnp.float32)
    asd = jnp.dot(h2, a_ref[...], preferred_element_type=jnp.float32)
    rows = i * BLK + lax.broadcasted_iota(jnp.int32, (BLK, 1), 0)
    valid = rows < N
    padrow = jnp.where(
        lax.broadcasted_iota(jnp.int32, (1, 16), 1) == 0, NEG_BIG, 0.0)
    o_ref[:, 0:64] = jnp.where(valid, h2, 0.0)
    o_ref[:, 64:80] = jnp.where(valid, asd, padrow)


def _stage2(p0, p1, W2, A2, R, b1, scale, shift):
    return pl.pallas_call(
        _stage2_body,
        out_shape=jax.ShapeDtypeStruct((NR, ROW2), jnp.float32),
        grid=(NR // BLK,),
        in_specs=[
            pl.BlockSpec((BLK, ROW1), lambda i: (i, 0)),
            pl.BlockSpec((BLK, ROW1), lambda i: (i, 0)),
            pl.BlockSpec((128, 64), lambda i: (0, 0)),
            pl.BlockSpec((64, 16), lambda i: (0, 0)),
            pl.BlockSpec((8, 128), lambda i: (0, 0)),
            pl.BlockSpec((1, 128), lambda i: (0, 0)),
            pl.BlockSpec((1, 128), lambda i: (0, 0)),
            pl.BlockSpec((1, 128), lambda i: (0, 0)),
        ],
        out_specs=pl.BlockSpec((BLK, ROW2), lambda i: (i, 0)),
    )(p0, p1, W2, A2, R, b1, scale, shift)


def _shift2_body(t_ref, o_ref):
    t = t_ref[:, 64:80]
    asv, adv = t[:, 0:1], t[:, 1:2]
    m = jnp.max(asv, axis=0, keepdims=True)
    tt = m + adv
    c = jnp.where(tt >= 0, tt, NEG * tt)
    z = jnp.zeros((t.shape[0], 15), dtype=jnp.float32)
    o_ref[...] = jnp.concatenate([adv, z, c, z], axis=1)


def _shift2(table2):
    return pl.pallas_call(
        _shift2_body,
        out_shape=jax.ShapeDtypeStruct((NR, 32), jnp.float32),
        grid=(1,),
        in_specs=[pl.BlockSpec((NR, ROW2), lambda i: (0, 0))],
        out_specs=pl.BlockSpec((NR, 32), lambda i: (0, 0)),
    )(table2)


def _stage3_body(q0_ref, q1_ref, b2_ref, o_ref):
    s = q0_ref[...] + q1_ref[...]
    feats = s[:, :64]
    den = s[:, 64:65]
    o = feats / (den + 1e-16) + b2_ref[...]
    m = jnp.max(o, axis=1, keepdims=True)
    l = o - m
    o_ref[...] = l - jnp.log(jnp.sum(jnp.exp(l), axis=1, keepdims=True))


def _stage3(q0, q1, b2):
    return pl.pallas_call(
        _stage3_body,
        out_shape=jax.ShapeDtypeStruct((NR, D_OUT), jnp.float32),
        grid=(NR // BLK,),
        in_specs=[
            pl.BlockSpec((BLK, ROW2), lambda i: (i, 0)),
            pl.BlockSpec((BLK, ROW2), lambda i: (i, 0)),
            pl.BlockSpec((1, 64), lambda i: (0, 0)),
        ],
        out_specs=pl.BlockSpec((BLK, D_OUT), lambda i: (i, 0)),
    )(q0, q1, b2)


def _make_edge_pass(roww, ch, mask_width, ep):
    nch = ep // (NW * CHK)
    ngrp = nch // GRP
    npair = GRP // 2
    nfeat = (roww - 16) // 16
    zslices = (NR // 16) // CHK
    mesh = plsc.VectorSubcoreMesh(core_axis_name="c", subcore_axis_name="s")

    @functools.partial(
        pl.kernel, mesh=mesh,
        compiler_params=pltpu.CompilerParams(use_tc_tiling_on_sc=False),
        out_type=jax.ShapeDtypeStruct((2, NR, roww), jnp.float32),
        scratch_types=[
            pltpu.VMEM((GRP, CHK), jnp.int32),
            pltpu.VMEM((GRP, CHK), jnp.int32),
            pltpu.VMEM((CHK, roww), jnp.float32),
            pltpu.VMEM((CHK, roww), jnp.float32),
            pltpu.VMEM((CHK, 32), jnp.float32),
            pltpu.VMEM((CHK, 32), jnp.float32),
            pltpu.VMEM_SHARED((NR, roww), jnp.float32),
            pltpu.SemaphoreType.DMA,
            pltpu.SemaphoreType.DMA,
            pltpu.SemaphoreType.DMA,
            pltpu.SemaphoreType.DMA,
        ],
    )
    def edge_pass(t1, tA, srcr, dstr, out, sbuf, dbuf, gb0, gb1, ab0, ab1,
                  acc, sg0, sg1, sa0, sa1):
        cid = lax.axis_index("c")
        sid = lax.axis_index("s")
        wid = sid * 2 + cid
        lane = lax.iota(jnp.int32, 16)
        emask = lane < mask_width
        zero16 = jnp.zeros((16,), jnp.float32)

        def zrow(i, _):
            for j in range(roww // 16):
                gb0[i, pl.ds(j * 16, 16)] = zero16
            return 0
        lax.fori_loop(0, CHK, zrow, 0)
        base = sid * (NR // 16)

        def zcp(k, _):
            pltpu.sync_copy(gb0, acc.at[pl.ds(base + k * CHK, CHK)])
            return 0
        lax.fori_loop(0, zslices, zcp, 0)
        plsc.subcore_barrier()

        def start(l, gb, ab, sg, sa):
            pltpu.async_copy(t1.at[sbuf.at[l]], gb, sg)
            pltpu.async_copy(tA.at[dbuf.at[l]], ab, sa)

        def wait(gb, ab, sg, sa):
            pltpu.make_async_copy(t1.at[sbuf.at[0]], gb, sg).wait()
            pltpu.make_async_copy(tA.at[dbuf.at[0]], ab, sa).wait()

        def process(l, gb, ab):
            def edge(i, _):
                tail = gb[i, pl.ds(roww - 16, 16)]
                adv = ab[i, pl.ds(0, 16)]
                cv = ab[i, pl.ds(16, 16)]
                raw = tail + adv
                lk = jnp.where(raw >= 0, raw, NEG * raw)
                ev = jnp.exp(lk - cv)
                for j in range(nfeat):
                    hj = (j * 16) // ch
                    ej = lax.gather(
                        ev, jnp.full((16, 1), hj, dtype=jnp.int32),
                        lax.GatherDimensionNumbers(
                            offset_dims=(), collapsed_slice_dims=(0,),
                            start_index_map=(0,)),
                        (1,), mode=lax.GatherScatterMode.PROMISE_IN_BOUNDS)
                    gb[i, pl.ds(j * 16, 16)] = gb[i, pl.ds(j * 16, 16)] * ej
                gb[i, pl.ds(roww - 16, 16)] = jnp.where(emask, ev, zero16)
                return 0
            lax.fori_loop(0, CHK, edge, 0)
            pltpu.sync_copy(gb, acc.at[dbuf.at[l]], add=True)

        def group(g, _):
            cbase = wid * nch + g * GRP
            pltpu.sync_copy(srcr.at[pl.ds(cbase, GRP)], sbuf)
            pltpu.sync_copy(dstr.at[pl.ds(cbase, GRP)], dbuf)
            start(0, gb0, ab0, sg0, sa0)

            def pair(q, _):
                l0 = 2 * q
                start(l0 + 1, gb1, ab1, sg1, sa1)
                wait(gb0, ab0, sg0, sa0)
                process(l0, gb0, ab0)

                @pl.when(q < npair - 1)
                def _():
                    start(l0 + 2, gb0, ab0, sg0, sa0)
                wait(gb1, ab1, sg1, sa1)
                process(l0 + 1, gb1, ab1)
                return 0
            lax.fori_loop(0, npair, pair, 0)
            return 0
        lax.fori_loop(0, ngrp, group, 0)
        plsc.subcore_barrier()

        def ocp(k, _):
            pltpu.sync_copy(acc.at[pl.ds(base + k * CHK, CHK)],
                            out.at[cid, pl.ds(base + k * CHK, CHK)])
            return 0
        lax.fori_loop(0, zslices, ocp, 0)

    return edge_pass


def kernel(x, edge_index, W1, a_src1, a_dst1, b1, gamma, beta, run_mean,
           run_var, W2, a_src2, a_dst2, b2):
    e2 = edge_index.shape[1] + N
    epq = NW * CHK * GRP
    ep = ((e2 + epq - 1) // epq) * epq

    xp = jnp.pad(x, ((0, NR - N), (0, 0)))
    ar = jnp.arange(N, dtype=edge_index.dtype)
    ei = jnp.concatenate([edge_index, jnp.stack([ar, ar])], axis=1)
    src = jnp.pad(ei[0], (0, ep - e2), constant_values=N).reshape(
        ep // CHK, CHK)
    dst = jnp.pad(ei[1], (0, ep - e2), constant_values=N).reshape(
        ep // CHK, CHK)

    rows128 = jnp.arange(128)
    head_of = jnp.repeat(jnp.arange(8), 16)
    A_src = jnp.zeros((128, 8), jnp.float32).at[rows128, head_of].set(
        a_src1.reshape(128))
    A_dst = jnp.zeros((128, 8), jnp.float32).at[rows128, head_of].set(
        a_dst1.reshape(128))
    A1 = jnp.concatenate([A_src, A_dst], axis=1)
    A2 = jnp.concatenate(
        [a_src2.reshape(64, 1), a_dst2.reshape(64, 1),
         jnp.zeros((64, 14), jnp.float32)], axis=1)
    Rm = jnp.repeat(jnp.eye(8, dtype=jnp.float32), 16, axis=1)
    scale = (gamma / jnp.sqrt(run_var + 1e-5)).reshape(1, 128)
    shift = (beta - run_mean * scale[0]).reshape(1, 128)

    table1 = _stage1(xp, W1, A1)
    tA1 = _shift1(table1)
    acc1 = _make_edge_pass(ROW1, HID, HEADS, ep)(table1, tA1, src, dst)
    table2 = _stage2(acc1[0], acc1[1], W2, A2, Rm, b1.reshape(1, 128),
                     scale, shift)
    tA2 = _shift2(table2)
    acc2 = _make_edge_pass(ROW2, D_OUT, 1, ep)(table2, tA2, src, dst)
    out = _stage3(acc2[0], acc2[1], b2.reshape(1, 64))
    return out[:N]


# speedup vs baseline: 78.4019x; 1.2206x over previous
import functools

import jax
import jax.numpy as jnp
from jax import lax
from jax.experimental import pallas as pl
from jax.experimental.pallas import tpu as pltpu
from jax.experimental.pallas import tpu_sc as plsc

N = 10000
NR = 10240
D_IN = 128
HEADS = 8
HID = 16
D_OUT = 64
NEG = 0.2
ROW1 = 144
ROW2 = 80
CHK = 64
GRP = 54
NW = 32
BLK = 512
NEG_BIG = -1e30


def _stage1_body(x_ref, w_ref, a_ref, o_ref):
    i = pl.program_id(0)
    h = jnp.dot(x_ref[...], w_ref[...], preferred_element_type=jnp.float32)
    asd = jnp.dot(h, a_ref[...], preferred_element_type=jnp.float32)
    rows = i * BLK + lax.broadcasted_iota(jnp.int32, (BLK, 1), 0)
    valid = rows < N
    hm = jnp.where(valid, h, 0.0)
    asm = jnp.where(valid, asd[:, :8], NEG_BIG)
    adm = jnp.where(valid, asd[:, 8:], 0.0)
    o_ref[:, 0:128] = hm
    o_ref[:, 128:144] = jnp.concatenate([asm, adm], axis=1)


def _stage1(xp, W1, A1):
    return pl.pallas_call(
        _stage1_body,
        out_shape=jax.ShapeDtypeStruct((NR, ROW1), jnp.float32),
        grid=(NR // BLK,),
        in_specs=[
            pl.BlockSpec((BLK, D_IN), lambda i: (i, 0)),
            pl.BlockSpec((D_IN, D_IN), lambda i: (0, 0)),
            pl.BlockSpec((D_IN, 16), lambda i: (0, 0)),
        ],
        out_specs=pl.BlockSpec((BLK, ROW1), lambda i: (i, 0)),
    )(xp, W1, A1)


def _shift1_body(t_ref, o_ref):
    t = t_ref[:, 128:144]
    asv, adv = t[:, :8], t[:, 8:]
    m = jnp.max(asv, axis=0, keepdims=True)
    tt = m + adv
    c = jnp.where(tt >= 0, tt, NEG * tt)
    z = jnp.zeros_like(adv)
    o_ref[...] = jnp.concatenate([adv, z, c, z], axis=1)


def _shift1(table1):
    return pl.pallas_call(
        _shift1_body,
        out_shape=jax.ShapeDtypeStruct((NR, 32), jnp.float32),
        grid=(1,),
        in_specs=[pl.BlockSpec((NR, ROW1), lambda i: (0, 0))],
        out_specs=pl.BlockSpec((NR, 32), lambda i: (0, 0)),
    )(table1)


def _stage2_body(p0_ref, p1_ref, w_ref, a_ref, r_ref, b1_ref, sc_ref, sh_ref,
                 o_ref):
    i = pl.program_id(0)
    s = p0_ref[...] + p1_ref[...]
    feats = s[:, :128]
    den = s[:, 128:136]
    denb = jnp.dot(den, r_ref[...], preferred_element_type=jnp.float32)
    h1 = feats / (denb + 1e-16) + b1_ref[...]
    h1 = h1 * sc_ref[...] + sh_ref[...]
    h1 = jnp.maximum(h1, 0.0)
    h2 = jnp.dot(h1, w_ref[...], preferred_element_type=jnp.float32)
    asd = jnp.dot(h2, a_ref[...], preferred_element_type=jnp.float32)
    rows = i * BLK + lax.broadcasted_iota(jnp.int32, (BLK, 1), 0)
    valid = rows < N
    padrow = jnp.where(
        lax.broadcasted_iota(jnp.int32, (1, 16), 1) == 0, NEG_BIG, 0.0)
    o_ref[:, 0:64] = jnp.where(valid, h2, 0.0)
    o_ref[:, 64:80] = jnp.where(valid, asd, padrow)


def _stage2(p0, p1, W2, A2, R, b1, scale, shift):
    return pl.pallas_call(
        _stage2_body,
        out_shape=jax.ShapeDtypeStruct((NR, ROW2), jnp.float32),
        grid=(NR // BLK,),
        in_specs=[
            pl.BlockSpec((BLK, ROW1), lambda i: (i, 0)),
            pl.BlockSpec((BLK, ROW1), lambda i: (i, 0)),
            pl.BlockSpec((128, 64), lambda i: (0, 0)),
            pl.BlockSpec((64, 16), lambda i: (0, 0)),
            pl.BlockSpec((8, 128), lambda i: (0, 0)),
            pl.BlockSpec((1, 128), lambda i: (0, 0)),
            pl.BlockSpec((1, 128), lambda i: (0, 0)),
            pl.BlockSpec((1, 128), lambda i: (0, 0)),
        ],
        out_specs=pl.BlockSpec((BLK, ROW2), lambda i: (i, 0)),
    )(p0, p1, W2, A2, R, b1, scale, shift)


def _shift2_body(t_ref, o_ref):
    t = t_ref[:, 64:80]
    asv, adv = t[:, 0:1], t[:, 1:2]
    m = jnp.max(asv, axis=0, keepdims=True)
    tt = m + adv
    c = jnp.where(tt >= 0, tt, NEG * tt)
    z = jnp.zeros((t.shape[0], 15), dtype=jnp.float32)
    o_ref[...] = jnp.concatenate([adv, z, c, z], axis=1)


def _shift2(table2):
    return pl.pallas_call(
        _shift2_body,
        out_shape=jax.ShapeDtypeStruct((NR, 32), jnp.float32),
        grid=(1,),
        in_specs=[pl.BlockSpec((NR, ROW2), lambda i: (0, 0))],
        out_specs=pl.BlockSpec((NR, 32), lambda i: (0, 0)),
    )(table2)


def _stage3_body(q0_ref, q1_ref, b2_ref, o_ref):
    s = q0_ref[...] + q1_ref[...]
    feats = s[:, :64]
    den = s[:, 64:65]
    o = feats / (den + 1e-16) + b2_ref[...]
    m = jnp.max(o, axis=1, keepdims=True)
    l = o - m
    o_ref[...] = l - jnp.log(jnp.sum(jnp.exp(l), axis=1, keepdims=True))


def _stage3(q0, q1, b2):
    return pl.pallas_call(
        _stage3_body,
        out_shape=jax.ShapeDtypeStruct((NR, D_OUT), jnp.float32),
        grid=(NR // BLK,),
        in_specs=[
            pl.BlockSpec((BLK, ROW2), lambda i: (i, 0)),
            pl.BlockSpec((BLK, ROW2), lambda i: (i, 0)),
            pl.BlockSpec((1, 64), lambda i: (0, 0)),
        ],
        out_specs=pl.BlockSpec((BLK, D_OUT), lambda i: (i, 0)),
    )(q0, q1, b2)


def _make_edge_pass(roww, ch, mask_width, ep):
    nch = ep // (NW * CHK)
    ngrp = nch // GRP
    npair = GRP // 2
    nfeat = (roww - 16) // 16
    zslices = (NR // 16) // CHK
    mesh = plsc.VectorSubcoreMesh(core_axis_name="c", subcore_axis_name="s")

    @functools.partial(
        pl.kernel, mesh=mesh,
        compiler_params=pltpu.CompilerParams(use_tc_tiling_on_sc=False),
        out_type=jax.ShapeDtypeStruct((2, NR, roww), jnp.float32),
        scratch_types=[
            pltpu.VMEM((GRP, CHK), jnp.int32),
            pltpu.VMEM((GRP, CHK), jnp.int32),
            pltpu.VMEM((CHK, roww), jnp.float32),
            pltpu.VMEM((CHK, roww), jnp.float32),
            pltpu.VMEM((CHK, 32), jnp.float32),
            pltpu.VMEM((CHK, 32), jnp.float32),
            pltpu.VMEM_SHARED((NR, roww), jnp.float32),
            pltpu.SemaphoreType.DMA,
            pltpu.SemaphoreType.DMA,
            pltpu.SemaphoreType.DMA,
            pltpu.SemaphoreType.DMA,
            pltpu.SemaphoreType.DMA,
            pltpu.SemaphoreType.DMA,
        ],
    )
    def edge_pass(t1, tA, srcr, dstr, out, sbuf, dbuf, gb0, gb1, ab0, ab1,
                  acc, sg0, sg1, sa0, sa1, ss0, ss1):
        cid = lax.axis_index("c")
        sid = lax.axis_index("s")
        wid = sid * 2 + cid
        lane = lax.iota(jnp.int32, 16)
        emask = lane < mask_width
        zero16 = jnp.zeros((16,), jnp.float32)

        def zrow(i, _):
            for j in range(roww // 16):
                gb0[i, pl.ds(j * 16, 16)] = zero16
            return 0
        lax.fori_loop(0, CHK, zrow, 0)
        base = sid * (NR // 16)

        def zcp(k, _):
            pltpu.sync_copy(gb0, acc.at[pl.ds(base + k * CHK, CHK)])
            return 0
        lax.fori_loop(0, zslices, zcp, 0)
        plsc.subcore_barrier()

        def start(l, gb, ab, sg, sa):
            pltpu.async_copy(t1.at[sbuf.at[l]], gb, sg)
            pltpu.async_copy(tA.at[dbuf.at[l]], ab, sa)

        def wait(gb, ab, sg, sa):
            pltpu.make_async_copy(t1.at[sbuf.at[0]], gb, sg).wait()
            pltpu.make_async_copy(tA.at[dbuf.at[0]], ab, sa).wait()

        def compute(gb, ab):
            @plsc.parallel_loop(0, CHK, 1, unroll=4)
            def edge(i):
                tail = gb[i, pl.ds(roww - 16, 16)]
                adv = ab[i, pl.ds(0, 16)]
                cv = ab[i, pl.ds(16, 16)]
                raw = tail + adv
                lk = jnp.where(raw >= 0, raw, NEG * raw)
                ev = jnp.exp(lk - cv)
                for j in range(nfeat):
                    hj = (j * 16) // ch
                    ej = lax.gather(
                        ev, jnp.full((16, 1), hj, dtype=jnp.int32),
                        lax.GatherDimensionNumbers(
                            offset_dims=(), collapsed_slice_dims=(0,),
                            start_index_map=(0,)),
                        (1,), mode=lax.GatherScatterMode.PROMISE_IN_BOUNDS)
                    gb[i, pl.ds(j * 16, 16)] = gb[i, pl.ds(j * 16, 16)] * ej
                gb[i, pl.ds(roww - 16, 16)] = jnp.where(emask, ev, zero16)

        def scat_start(l, gb, ss):
            pltpu.async_copy(gb, acc.at[dbuf.at[l]], ss, add=True)

        def scat_wait(gb, ss):
            pltpu.make_async_copy(gb, acc.at[dbuf.at[0]], ss).wait()

        def group(g, _):
            cbase = wid * nch + g * GRP
            pltpu.sync_copy(srcr.at[pl.ds(cbase, GRP)], sbuf)
            pltpu.sync_copy(dstr.at[pl.ds(cbase, GRP)], dbuf)
            start(0, gb0, ab0, sg0, sa0)

            def pair(q, _):
                l0 = 2 * q

                @pl.when(q > 0)
                def _():
                    scat_wait(gb1, ss1)
                start(l0 + 1, gb1, ab1, sg1, sa1)
                wait(gb0, ab0, sg0, sa0)
                compute(gb0, ab0)
                scat_start(l0, gb0, ss0)
                wait(gb1, ab1, sg1, sa1)
                compute(gb1, ab1)
                scat_wait(gb0, ss0)

                @pl.when(q < npair - 1)
                def _():
                    start(l0 + 2, gb0, ab0, sg0, sa0)
                scat_start(l0 + 1, gb1, ss1)
                return 0
            lax.fori_loop(0, npair, pair, 0)
            scat_wait(gb1, ss1)
            return 0
        lax.fori_loop(0, ngrp, group, 0)
        plsc.subcore_barrier()

        def ocp(k, _):
            pltpu.sync_copy(acc.at[pl.ds(base + k * CHK, CHK)],
                            out.at[cid, pl.ds(base + k * CHK, CHK)])
            return 0
        lax.fori_loop(0, zslices, ocp, 0)

    return edge_pass


def kernel(x, edge_index, W1, a_src1, a_dst1, b1, gamma, beta, run_mean,
           run_var, W2, a_src2, a_dst2, b2):
    e2 = edge_index.shape[1] + N
    epq = NW * CHK * GRP
    ep = ((e2 + epq - 1) // epq) * epq

    xp = jnp.pad(x, ((0, NR - N), (0, 0)))
    ar = jnp.arange(N, dtype=edge_index.dtype)
    ei = jnp.concatenate([edge_index, jnp.stack([ar, ar])], axis=1)
    src = jnp.pad(ei[0], (0, ep - e2), constant_values=N).reshape(
        ep // CHK, CHK)
    dst = jnp.pad(ei[1], (0, ep - e2), constant_values=N).reshape(
        ep // CHK, CHK)

    rows128 = jnp.arange(128)
    head_of = jnp.repeat(jnp.arange(8), 16)
    A_src = jnp.zeros((128, 8), jnp.float32).at[rows128, head_of].set(
        a_src1.reshape(128))
    A_dst = jnp.zeros((128, 8), jnp.float32).at[rows128, head_of].set(
        a_dst1.reshape(128))
    A1 = jnp.concatenate([A_src, A_dst], axis=1)
    A2 = jnp.concatenate(
        [a_src2.reshape(64, 1), a_dst2.reshape(64, 1),
         jnp.zeros((64, 14), jnp.float32)], axis=1)
    Rm = jnp.repeat(jnp.eye(8, dtype=jnp.float32), 16, axis=1)
    scale = (gamma / jnp.sqrt(run_var + 1e-5)).reshape(1, 128)
    shift = (beta - run_mean * scale[0]).reshape(1, 128)

    table1 = _stage1(xp, W1, A1)
    tA1 = _shift1(table1)
    acc1 = _make_edge_pass(ROW1, HID, HEADS, ep)(table1, tA1, src, dst)
    table2 = _stage2(acc1[0], acc1[1], W2, A2, Rm, b1.reshape(1, 128),
                     scale, shift)
    tA2 = _shift2(table2)
    acc2 = _make_edge_pass(ROW2, D_OUT, 1, ep)(table2, tA2, src, dst)
    out = _stage3(acc2[0], acc2[1], b2.reshape(1, 64))
    return out[:N]


# speedup vs baseline: 78.6802x; 1.0035x over previous
import functools

import jax
import jax.numpy as jnp
from jax import lax
from jax.experimental import pallas as pl
from jax.experimental.pallas import tpu as pltpu
from jax.experimental.pallas import tpu_sc as plsc

N = 10000
NR = 10240
D_IN = 128
HEADS = 8
HID = 16
D_OUT = 64
NEG = 0.2
ROW1 = 144
ROW2 = 80
CHK = 64
GRP = 54
NW = 32
BLK = 512
NEG_BIG = -1e30


def _stage1_body(x_ref, w_ref, a_ref, o_ref):
    i = pl.program_id(0)
    h = jnp.dot(x_ref[...], w_ref[...], preferred_element_type=jnp.float32)
    asd = jnp.dot(h, a_ref[...], preferred_element_type=jnp.float32)
    rows = i * BLK + lax.broadcasted_iota(jnp.int32, (BLK, 1), 0)
    valid = rows < N
    hm = jnp.where(valid, h, 0.0)
    asm = jnp.where(valid, asd[:, :8], NEG_BIG)
    adm = jnp.where(valid, asd[:, 8:], 0.0)
    o_ref[:, 0:128] = hm
    o_ref[:, 128:144] = jnp.concatenate([asm, adm], axis=1)


def _stage1(xp, W1, A1):
    return pl.pallas_call(
        _stage1_body,
        out_shape=jax.ShapeDtypeStruct((NR, ROW1), jnp.float32),
        grid=(NR // BLK,),
        in_specs=[
            pl.BlockSpec((BLK, D_IN), lambda i: (i, 0)),
            pl.BlockSpec((D_IN, D_IN), lambda i: (0, 0)),
            pl.BlockSpec((D_IN, 16), lambda i: (0, 0)),
        ],
        out_specs=pl.BlockSpec((BLK, ROW1), lambda i: (i, 0)),
    )(xp, W1, A1)


def _shift1_body(t_ref, o_ref):
    t = t_ref[:, 128:144]
    asv, adv = t[:, :8], t[:, 8:]
    m = jnp.max(asv, axis=0, keepdims=True)
    tt = m + adv
    c = jnp.where(tt >= 0, tt, NEG * tt)
    z = jnp.zeros_like(adv)
    o_ref[...] = jnp.concatenate([adv, z, c, z], axis=1)


def _shift1(table1):
    return pl.pallas_call(
        _shift1_body,
        out_shape=jax.ShapeDtypeStruct((NR, 32), jnp.float32),
        grid=(1,),
        in_specs=[pl.BlockSpec((NR, ROW1), lambda i: (0, 0))],
        out_specs=pl.BlockSpec((NR, 32), lambda i: (0, 0)),
    )(table1)


def _stage2_body(p0_ref, p1_ref, w_ref, a_ref, r_ref, b1_ref, sc_ref, sh_ref,
                 o_ref):
    i = pl.program_id(0)
    s = p0_ref[...] + p1_ref[...]
    feats = s[:, :128]
    den = s[:, 128:136]
    denb = jnp.dot(den, r_ref[...], preferred_element_type=jnp.float32)
    h1 = feats / (denb + 1e-16) + b1_ref[...]
    h1 = h1 * sc_ref[...] + sh_ref[...]
    h1 = jnp.maximum(h1, 0.0)
    h2 = jnp.dot(h1, w_ref[...], preferred_element_type=jnp.float32)
    asd = jnp.dot(h2, a_ref[...], preferred_element_type=jnp.float32)
    rows = i * BLK + lax.broadcasted_iota(jnp.int32, (BLK, 1), 0)
    valid = rows < N
    padrow = jnp.where(
        lax.broadcasted_iota(jnp.int32, (1, 16), 1) == 0, NEG_BIG, 0.0)
    o_ref[:, 0:64] = jnp.where(valid, h2, 0.0)
    o_ref[:, 64:80] = jnp.where(valid, asd, padrow)


def _stage2(p0, p1, W2, A2, R, b1, scale, shift):
    return pl.pallas_call(
        _stage2_body,
        out_shape=jax.ShapeDtypeStruct((NR, ROW2), jnp.float32),
        grid=(NR // BLK,),
        in_specs=[
            pl.BlockSpec((BLK, ROW1), lambda i: (i, 0)),
            pl.BlockSpec((BLK, ROW1), lambda i: (i, 0)),
            pl.BlockSpec((128, 64), lambda i: (0, 0)),
            pl.BlockSpec((64, 16), lambda i: (0, 0)),
            pl.BlockSpec((8, 128), lambda i: (0, 0)),
            pl.BlockSpec((1, 128), lambda i: (0, 0)),
            pl.BlockSpec((1, 128), lambda i: (0, 0)),
            pl.BlockSpec((1, 128), lambda i: (0, 0)),
        ],
        out_specs=pl.BlockSpec((BLK, ROW2), lambda i: (i, 0)),
    )(p0, p1, W2, A2, R, b1, scale, shift)


def _shift2_body(t_ref, o_ref):
    t = t_ref[:, 64:80]
    asv, adv = t[:, 0:1], t[:, 1:2]
    m = jnp.max(asv, axis=0, keepdims=True)
    tt = m + adv
    c = jnp.where(tt >= 0, tt, NEG * tt)
    z = jnp.zeros((t.shape[0], 15), dtype=jnp.float32)
    o_ref[...] = jnp.concatenate([adv, z, c, z], axis=1)


def _shift2(table2):
    return pl.pallas_call(
        _shift2_body,
        out_shape=jax.ShapeDtypeStruct((NR, 32), jnp.float32),
        grid=(1,),
        in_specs=[pl.BlockSpec((NR, ROW2), lambda i: (0, 0))],
        out_specs=pl.BlockSpec((NR, 32), lambda i: (0, 0)),
    )(table2)


def _stage3_body(q0_ref, q1_ref, b2_ref, o_ref):
    s = q0_ref[...] + q1_ref[...]
    feats = s[:, :64]
    den = s[:, 64:65]
    o = feats / (den + 1e-16) + b2_ref[...]
    m = jnp.max(o, axis=1, keepdims=True)
    l = o - m
    o_ref[...] = l - jnp.log(jnp.sum(jnp.exp(l), axis=1, keepdims=True))


def _stage3(q0, q1, b2):
    return pl.pallas_call(
        _stage3_body,
        out_shape=jax.ShapeDtypeStruct((NR, D_OUT), jnp.float32),
        grid=(NR // BLK,),
        in_specs=[
            pl.BlockSpec((BLK, ROW2), lambda i: (i, 0)),
            pl.BlockSpec((BLK, ROW2), lambda i: (i, 0)),
            pl.BlockSpec((1, 64), lambda i: (0, 0)),
        ],
        out_specs=pl.BlockSpec((BLK, D_OUT), lambda i: (i, 0)),
    )(q0, q1, b2)


def _make_edge_pass(roww, ch, mask_width, ep):
    nch = ep // (NW * CHK)
    ngrp = nch // GRP
    npair = GRP // 2
    nfeat = (roww - 16) // 16
    zslices = (NR // 16) // CHK
    mesh = plsc.VectorSubcoreMesh(core_axis_name="c", subcore_axis_name="s")

    @functools.partial(
        pl.kernel, mesh=mesh,
        compiler_params=pltpu.CompilerParams(use_tc_tiling_on_sc=False),
        out_type=jax.ShapeDtypeStruct((2, NR, roww), jnp.float32),
        scratch_types=[
            pltpu.VMEM((GRP, CHK), jnp.int32),
            pltpu.VMEM((GRP, CHK), jnp.int32),
            pltpu.VMEM((CHK, roww), jnp.float32),
            pltpu.VMEM((CHK, roww), jnp.float32),
            pltpu.VMEM((CHK, 32), jnp.float32),
            pltpu.VMEM((CHK, 32), jnp.float32),
            pltpu.VMEM_SHARED((NR, roww), jnp.float32),
            pltpu.SemaphoreType.DMA,
            pltpu.SemaphoreType.DMA,
            pltpu.SemaphoreType.DMA,
            pltpu.SemaphoreType.DMA,
            pltpu.SemaphoreType.DMA,
            pltpu.SemaphoreType.DMA,
        ],
    )
    def edge_pass(t1, tA, srcr, dstr, out, sbuf, dbuf, gb0, gb1, ab0, ab1,
                  acc, sg0, sg1, sa0, sa1, ss0, ss1):
        cid = lax.axis_index("c")
        sid = lax.axis_index("s")
        wid = sid * 2 + cid
        lane = lax.iota(jnp.int32, 16)
        emask = lane < mask_width
        zero16 = jnp.zeros((16,), jnp.float32)

        def zrow(i, _):
            for j in range(roww // 16):
                gb0[i, pl.ds(j * 16, 16)] = zero16
            return 0
        lax.fori_loop(0, CHK, zrow, 0)
        base = sid * (NR // 16)

        def zcp(k, _):
            pltpu.sync_copy(gb0, acc.at[pl.ds(base + k * CHK, CHK)])
            return 0
        lax.fori_loop(0, zslices, zcp, 0)
        plsc.subcore_barrier()

        def start(l, gb, ab, sg, sa):
            pltpu.async_copy(t1.at[sbuf.at[l]], gb, sg)
            pltpu.async_copy(tA.at[dbuf.at[l]], ab, sa)

        def wait(gb, ab, sg, sa):
            pltpu.make_async_copy(t1.at[sbuf.at[0]], gb, sg).wait()
            pltpu.make_async_copy(tA.at[dbuf.at[0]], ab, sa).wait()

        dup_idx = (lane % mask_width).reshape(16, 1)

        def compute(gb, ab):
            @plsc.parallel_loop(0, CHK, 1, unroll=4)
            def edge(i):
                tail = gb[i, pl.ds(roww - 16, 16)]
                adv = ab[i, pl.ds(0, 16)]
                cv = ab[i, pl.ds(16, 16)]
                raw = tail + adv
                lk = jnp.where(raw >= 0, raw, NEG * raw)
                ev = jnp.exp(lk - cv)
                ed = lax.gather(
                    ev, dup_idx,
                    lax.GatherDimensionNumbers(
                        offset_dims=(), collapsed_slice_dims=(0,),
                        start_index_map=(0,)),
                    (1,), mode=lax.GatherScatterMode.PROMISE_IN_BOUNDS)
                for j in range(nfeat):
                    gb[i, pl.ds(j * 16, 16)] = gb[i, pl.ds(j * 16, 16)] * ed
                gb[i, pl.ds(roww - 16, 16)] = jnp.where(emask, ev, zero16)

        def scat_start(l, gb, ss):
            pltpu.async_copy(gb, acc.at[dbuf.at[l]], ss, add=True)

        def scat_wait(gb, ss):
            pltpu.make_async_copy(gb, acc.at[dbuf.at[0]], ss).wait()

        def group(g, _):
            cbase = wid * nch + g * GRP
            pltpu.sync_copy(srcr.at[pl.ds(cbase, GRP)], sbuf)
            pltpu.sync_copy(dstr.at[pl.ds(cbase, GRP)], dbuf)
            start(0, gb0, ab0, sg0, sa0)

            def pair(q, _):
                l0 = 2 * q

                @pl.when(q > 0)
                def _():
                    scat_wait(gb1, ss1)
                start(l0 + 1, gb1, ab1, sg1, sa1)
                wait(gb0, ab0, sg0, sa0)
                compute(gb0, ab0)
                scat_start(l0, gb0, ss0)
                wait(gb1, ab1, sg1, sa1)
                compute(gb1, ab1)
                scat_wait(gb0, ss0)

                @pl.when(q < npair - 1)
                def _():
                    start(l0 + 2, gb0, ab0, sg0, sa0)
                scat_start(l0 + 1, gb1, ss1)
                return 0
            lax.fori_loop(0, npair, pair, 0)
            scat_wait(gb1, ss1)
            return 0
        lax.fori_loop(0, ngrp, group, 0)
        plsc.subcore_barrier()

        def ocp(k, _):
            pltpu.sync_copy(acc.at[pl.ds(base + k * CHK, CHK)],
                            out.at[cid, pl.ds(base + k * CHK, CHK)])
            return 0
        lax.fori_loop(0, zslices, ocp, 0)

    return edge_pass


def kernel(x, edge_index, W1, a_src1, a_dst1, b1, gamma, beta, run_mean,
           run_var, W2, a_src2, a_dst2, b2):
    e2 = edge_index.shape[1] + N
    epq = NW * CHK * GRP
    ep = ((e2 + epq - 1) // epq) * epq

    xp = jnp.pad(x, ((0, NR - N), (0, 0)))
    ar = jnp.arange(N, dtype=edge_index.dtype)
    ei = jnp.concatenate([edge_index, jnp.stack([ar, ar])], axis=1)
    src = jnp.pad(ei[0], (0, ep - e2), constant_values=N).reshape(
        ep // CHK, CHK)
    dst = jnp.pad(ei[1], (0, ep - e2), constant_values=N).reshape(
        ep // CHK, CHK)

    rows128 = jnp.arange(128)
    head_of = jnp.repeat(jnp.arange(8), 16)
    A_src = jnp.zeros((128, 8), jnp.float32).at[rows128, head_of].set(
        a_src1.reshape(128))
    A_dst = jnp.zeros((128, 8), jnp.float32).at[rows128, head_of].set(
        a_dst1.reshape(128))
    pvec = (rows128 % 8) * 16 + rows128 // 8
    W1p = W1[:, pvec]
    A1 = jnp.concatenate([A_src, A_dst], axis=1)[pvec, :]
    A2 = jnp.concatenate(
        [a_src2.reshape(64, 1), a_dst2.reshape(64, 1),
         jnp.zeros((64, 14), jnp.float32)], axis=1)
    Rm = (rows128[None, :] % 8 == jnp.arange(8)[:, None]).astype(jnp.float32)
    W2p = W2[pvec, :]
    scale = gamma / jnp.sqrt(run_var + 1e-5)
    shift = beta - run_mean * scale

    table1 = _stage1(xp, W1p, A1)
    tA1 = _shift1(table1)
    acc1 = _make_edge_pass(ROW1, HID, HEADS, ep)(table1, tA1, src, dst)
    table2 = _stage2(acc1[0], acc1[1], W2p, A2, Rm, b1[pvec].reshape(1, 128),
                     scale[pvec].reshape(1, 128), shift[pvec].reshape(1, 128))
    tA2 = _shift2(table2)
    acc2 = _make_edge_pass(ROW2, D_OUT, 1, ep)(table2, tA2, src, dst)
    out = _stage3(acc2[0], acc2[1], b2.reshape(1, 64))
    return out[:N]
